# baseline (speedup 1.0000x reference)
import functools

import jax
import jax.numpy as jnp
from jax import lax
from jax.experimental import pallas as pl
from jax.experimental.pallas import tpu as pltpu
from jax.experimental.pallas import tpu_sc as plsc

B, H, W_IN, C = 8, 112, 112, 96
OUT_H, OUT_W = 2 * H, 2 * W_IN
N = B * H * W_IN * C
M = B * OUT_H * OUT_W * C

NTILE = 16
NCHUNK = 24
CHUNK = M // NCHUNK
PAD = 256
SLICE = CHUNK // NTILE
TS = N // NTILE
WIN = 4704
NWIN = TS // WIN
VPW = WIN // 16
UNROLL = 14


def _sc_body(idx_hbm, upd_hbm, zeros_hbm, out_hbm,
             ib0, ib1, ib2, vb0, vb1, vb2,
             acc, in0, in1, in2, sc0, sc1, sc2):
    c = lax.axis_index("c")
    s = lax.axis_index("s")

    iota = lax.iota(jnp.int32, 16)
    dump_u = plsc.bitcast(CHUNK + 8 * iota, jnp.uint32)

    sets = ((ib0, vb0, in0, sc0), (ib1, vb1, in1, sc1), (ib2, vb2, in2, sc2))

    def _issue_in(w, t):
        ib, vb, insem, _ = sets[t]
        src = s * TS + w * WIN
        pltpu.async_copy(idx_hbm.at[pl.ds(src, WIN)], ib, insem)
        pltpu.async_copy(upd_hbm.at[pl.ds(src, WIN)], vb, insem)

    def _wait_in(w, t):
        ib, vb, insem, _ = sets[t]
        src = s * TS + w * WIN
        pltpu.make_async_copy(idx_hbm.at[pl.ds(src, WIN)], ib, insem).wait()
        pltpu.make_async_copy(upd_hbm.at[pl.ds(src, WIN)], vb, insem).wait()

    def _drain_scat(t):
        ib, vb, _, scsem = sets[t]
        pltpu.make_async_copy(vb, acc.at[ib], scsem).wait()

    def _chunk(k, carry):
        chunk_id = 2 * k + c

        @pl.when(chunk_id < NCHUNK)
        def _():
            base = chunk_id * CHUNK
            base_vec = jnp.full((16,), 0, jnp.int32) + base
            for t in range(3):
                _issue_in(t, t)
            pltpu.sync_copy(zeros_hbm, acc.at[pl.ds(s * SLICE, SLICE)])
            plsc.subcore_barrier()

            def _step(w, t):
                ib, vb, _, scsem = sets[t]
                _wait_in(w, t)

                def _vecs(j, vcarry):
                    for u_ in range(UNROLL):
                        v = j * UNROLL + u_
                        u = plsc.bitcast(ib[pl.ds(v * 16, 16)] - base_vec,
                                         jnp.uint32)
                        off = jnp.minimum(u, dump_u + u_)
                        ib[pl.ds(v * 16, 16)] = plsc.bitcast(off, jnp.int32)
                    return vcarry

                lax.fori_loop(0, VPW // UNROLL, _vecs, 0)
                pltpu.async_copy(vb, acc.at[ib], scsem, add=True)

                tp = (t + 2) % 3

                @pl.when(w >= 1)
                def _():
                    _drain_scat(tp)

                    @pl.when(w + 2 < NWIN)
                    def _():
                        _issue_in(w + 2, tp)

            def _group(j, wcarry):
                for t in range(3):
                    _step(3 * j + t, t)
                return wcarry

            lax.fori_loop(0, NWIN // 3, _group, 0)
            for w_tail in range(3 * (NWIN // 3), NWIN):
                _step(jnp.int32(w_tail), w_tail % 3)
            _drain_scat((NWIN - 1) % 3)
            plsc.subcore_barrier()
            pltpu.sync_copy(
                acc.at[pl.ds(s * SLICE, SLICE)],
                out_hbm.at[pl.ds(base + s * SLICE, SLICE)],
            )

        return carry

    lax.fori_loop(0, (NCHUNK + 1) // 2, _chunk, 0)


@functools.partial(
    pl.kernel,
    mesh=plsc.VectorSubcoreMesh(core_axis_name="c", subcore_axis_name="s"),
    out_type=jax.ShapeDtypeStruct((M,), jnp.float32),
    scratch_types=[
        pltpu.VMEM((WIN,), jnp.int32),
        pltpu.VMEM((WIN,), jnp.int32),
        pltpu.VMEM((WIN,), jnp.int32),
        pltpu.VMEM((WIN,), jnp.float32),
        pltpu.VMEM((WIN,), jnp.float32),
        pltpu.VMEM((WIN,), jnp.float32),
        pltpu.VMEM_SHARED((CHUNK + PAD,), jnp.float32),
        pltpu.SemaphoreType.DMA,
        pltpu.SemaphoreType.DMA,
        pltpu.SemaphoreType.DMA,
        pltpu.SemaphoreType.DMA,
        pltpu.SemaphoreType.DMA,
        pltpu.SemaphoreType.DMA,
    ],
)
def _scatter_add(idx_hbm, upd_hbm, zeros_hbm, out_hbm,
                 ib0, ib1, ib2, vb0, vb1, vb2,
                 acc, in0, in1, in2, sc0, sc1, sc2):
    _sc_body(idx_hbm, upd_hbm, zeros_hbm, out_hbm,
             ib0, ib1, ib2, vb0, vb1, vb2,
             acc, in0, in1, in2, sc0, sc1, sc2)


@jax.jit
def kernel(updates, mask):
    idx = mask.reshape(-1).astype(jnp.int32)
    upd = updates.reshape(-1)
    zeros = jnp.zeros((SLICE,), jnp.float32)
    flat = _scatter_add(idx, upd, zeros)
    return flat.reshape(-1, OUT_H, OUT_W, C)


# speedup vs baseline: 1.0703x; 1.0703x over previous
import functools

import jax
import jax.numpy as jnp
from jax import lax
from jax.experimental import pallas as pl
from jax.experimental.pallas import tpu as pltpu
from jax.experimental.pallas import tpu_sc as plsc

B, H, W_IN, C = 8, 112, 112, 96
OUT_H, OUT_W = 2 * H, 2 * W_IN
N = B * H * W_IN * C
M = B * OUT_H * OUT_W * C

NTILE = 16
CHUNK = 7 * (1 << 18)
NCHUNK = M // CHUNK
PAD = 256
SLICE = CHUNK // NTILE
TS = N // NTILE
WIN = 1792
NWIN = TS // WIN
VPW = WIN // 16
UNROLL = 16
NSET = 4


def _sc_body(idx_hbm, upd_hbm, zeros_hbm, out_hbm, *scratch):
    ibs = scratch[0:4]
    vbs = scratch[4:8]
    acc = scratch[8]
    insems = scratch[9:13]
    scsems = scratch[13:17]

    c = lax.axis_index("c")
    s = lax.axis_index("s")

    iota = lax.iota(jnp.int32, 16)
    dump_u = plsc.bitcast(CHUNK + 8 * iota, jnp.uint32)

    def _issue_in(w, t):
        src = s * TS + w * WIN
        pltpu.async_copy(idx_hbm.at[pl.ds(src, WIN)], ibs[t], insems[t])
        pltpu.async_copy(upd_hbm.at[pl.ds(src, WIN)], vbs[t], insems[t])

    def _wait_in(w, t):
        src = s * TS + w * WIN
        pltpu.make_async_copy(idx_hbm.at[pl.ds(src, WIN)], ibs[t],
                              insems[t]).wait()
        pltpu.make_async_copy(upd_hbm.at[pl.ds(src, WIN)], vbs[t],
                              insems[t]).wait()

    def _drain_scat(t):
        pltpu.make_async_copy(vbs[t], acc.at[ibs[t]], scsems[t]).wait()

    def _chunk(k, carry):
        chunk_id = 2 * k + c

        @pl.when(chunk_id < NCHUNK)
        def _():
            base = chunk_id * CHUNK
            base_vec = jnp.full((16,), 0, jnp.int32) + base
            _issue_in(0, 0)
            _issue_in(1, 1)
            pltpu.sync_copy(zeros_hbm, acc.at[pl.ds(s * SLICE, SLICE)])
            plsc.subcore_barrier()

            def _step(w, t):
                ib = ibs[t]
                _wait_in(w, t)

                def _vecs(j, vcarry):
                    for u_ in range(UNROLL):
                        v = j * UNROLL + u_
                        u = plsc.bitcast(ib[pl.ds(v * 16, 16)] - base_vec,
                                         jnp.uint32)
                        off = jnp.minimum(u, dump_u + u_)
                        ib[pl.ds(v * 16, 16)] = plsc.bitcast(off, jnp.int32)
                    return vcarry

                lax.fori_loop(0, VPW // UNROLL, _vecs, 0)
                pltpu.async_copy(vbs[t], acc.at[ib], scsems[t], add=True)

                tn = (t + 2) % NSET

                @pl.when(w >= 2)
                def _():
                    _drain_scat(tn)

                @pl.when(w + 2 < NWIN)
                def _():
                    _issue_in(w + 2, tn)

            def _group(j, wcarry):
                for t in range(NSET):
                    _step(NSET * j + t, t)
                return wcarry

            lax.fori_loop(0, NWIN // NSET, _group, 0)
            _drain_scat((NWIN - 2) % NSET)
            _drain_scat((NWIN - 1) % NSET)
            plsc.subcore_barrier()
            pltpu.sync_copy(
                acc.at[pl.ds(s * SLICE, SLICE)],
                out_hbm.at[pl.ds(base + s * SLICE, SLICE)],
            )

        return carry

    lax.fori_loop(0, (NCHUNK + 1) // 2, _chunk, 0)


@functools.partial(
    pl.kernel,
    mesh=plsc.VectorSubcoreMesh(core_axis_name="c", subcore_axis_name="s"),
    out_type=jax.ShapeDtypeStruct((M,), jnp.float32),
    scratch_types=[
        pltpu.VMEM((WIN,), jnp.int32),
        pltpu.VMEM((WIN,), jnp.int32),
        pltpu.VMEM((WIN,), jnp.int32),
        pltpu.VMEM((WIN,), jnp.int32),
        pltpu.VMEM((WIN,), jnp.float32),
        pltpu.VMEM((WIN,), jnp.float32),
        pltpu.VMEM((WIN,), jnp.float32),
        pltpu.VMEM((WIN,), jnp.float32),
        pltpu.VMEM_SHARED((CHUNK + PAD,), jnp.float32),
        pltpu.SemaphoreType.DMA,
        pltpu.SemaphoreType.DMA,
        pltpu.SemaphoreType.DMA,
        pltpu.SemaphoreType.DMA,
        pltpu.SemaphoreType.DMA,
        pltpu.SemaphoreType.DMA,
        pltpu.SemaphoreType.DMA,
        pltpu.SemaphoreType.DMA,
    ],
)
def _scatter_add(idx_hbm, upd_hbm, zeros_hbm, out_hbm, *scratch):
    _sc_body(idx_hbm, upd_hbm, zeros_hbm, out_hbm, *scratch)


@jax.jit
def kernel(updates, mask):
    idx = mask.reshape(-1).astype(jnp.int32)
    upd = updates.reshape(-1)
    zeros = jnp.zeros((SLICE,), jnp.float32)
    flat = _scatter_add(idx, upd, zeros)
    return flat.reshape(-1, OUT_H, OUT_W, C)
